# baseline (speedup 1.0000x reference)
import functools

import jax
import jax.numpy as jnp
from jax import lax
from jax.experimental import pallas as pl
from jax.experimental.pallas import tpu as pltpu
from jax.experimental.pallas import tpu_sc as plsc

D = 128
NW = 32
GB = 4


def _build(B, H):
    Hp = (H + 7) // 8 * 8
    rows_per_w = B // NW
    T = rows_per_w // GB
    K = T // 2

    mesh = plsc.VectorSubcoreMesh(core_axis_name="c", subcore_axis_name="s")

    @functools.partial(
        pl.kernel,
        mesh=mesh,
        out_type=jax.ShapeDtypeStruct((B * Hp, D), jnp.float32),
        scratch_types=[
            pltpu.VMEM((rows_per_w * Hp,), jnp.int32),
            pltpu.VMEM((2, GB * Hp, D), jnp.float32),
            pltpu.SemaphoreType.DMA,
            pltpu.SemaphoreType.DMA,
            pltpu.SemaphoreType.DMA,
            pltpu.SemaphoreType.DMA,
        ],
    )
    def gather_kernel(idx_hbm, table_hbm, out_hbm, idx_v, rows_v,
                      sg0, sg1, so0, so1):
        wid = lax.axis_index("s") * 2 + lax.axis_index("c")
        base = wid * rows_per_w
        sg = (sg0, sg1)
        so = (so0, so1)

        pltpu.sync_copy(idx_hbm.at[pl.ds(base * Hp, rows_per_w * Hp)], idx_v)

        def fire_g(t, b):
            return pltpu.async_copy(
                table_hbm.at[idx_v.at[pl.ds(t * GB * Hp, GB * Hp)]],
                rows_v.at[b], sg[b])

        def fire_o(t, b):
            pltpu.async_copy(
                rows_v.at[b],
                out_hbm.at[pl.ds((base + t * GB) * Hp, GB * Hp)], so[b])

        def drain_o(b):
            pltpu.make_async_copy(
                rows_v.at[b], out_hbm.at[pl.ds(base * Hp, GB * Hp)],
                so[b]).wait()

        def pair(t0, first):
            if not first:
                drain_o(0)
            h0 = fire_g(t0, 0)
            if not first:
                drain_o(1)
            h1 = fire_g(t0 + 1, 1)
            h0.wait()
            fire_o(t0, 0)
            h1.wait()
            fire_o(t0 + 1, 1)

        pair(0, True)

        def body(k, carry):
            pair(k * 2, False)
            return carry

        lax.fori_loop(1, K, body, 0)
        drain_o(0)
        drain_o(1)

    return gather_kernel


def kernel(chars, table):
    B, H = chars.shape
    Hp = (H + 7) // 8 * 8
    idx = jnp.pad(chars.astype(jnp.int32), ((0, 0), (0, Hp - H))).reshape(-1)
    out = _build(B, H)(idx, table)
    return out.reshape(B, Hp, D)[:, :H, :]


# speedup vs baseline: 7.0517x; 7.0517x over previous
import functools

import jax
import jax.numpy as jnp
from jax import lax
from jax.experimental import pallas as pl
from jax.experimental.pallas import tpu as pltpu
from jax.experimental.pallas import tpu_sc as plsc

D = 128
NW = 32
GB = 4


def _build(B, H):
    Hp = (H + 7) // 8 * 8
    rows_per_w = B // NW
    T = rows_per_w // GB
    K = T // 2

    mesh = plsc.VectorSubcoreMesh(core_axis_name="c", subcore_axis_name="s")

    @functools.partial(
        pl.kernel,
        mesh=mesh,
        out_type=jax.ShapeDtypeStruct((B * Hp, D), jnp.float32),
        scratch_types=[
            pltpu.VMEM((rows_per_w * Hp,), jnp.int32),
            pltpu.VMEM((2, GB * Hp, D), jnp.float32),
            pltpu.SemaphoreType.DMA,
            pltpu.SemaphoreType.DMA,
            pltpu.SemaphoreType.DMA,
            pltpu.SemaphoreType.DMA,
        ],
    )
    def gather_kernel(idx_hbm, table_hbm, out_hbm, idx_v, rows_v,
                      sg0, sg1, so0, so1):
        wid = lax.axis_index("s") * 2 + lax.axis_index("c")
        base = wid * rows_per_w
        sg = (sg0, sg1)
        so = (so0, so1)

        pltpu.sync_copy(idx_hbm.at[pl.ds(base * Hp, rows_per_w * Hp)], idx_v)

        def fire_g(t, b):
            return pltpu.async_copy(
                table_hbm.at[idx_v.at[pl.ds(t * GB * Hp, GB * Hp)]],
                rows_v.at[b], sg[b])

        def fire_o(t, b):
            pltpu.async_copy(
                rows_v.at[b],
                out_hbm.at[pl.ds((base + t * GB) * Hp, GB * Hp)], so[b])

        def drain_o(b):
            pltpu.make_async_copy(
                rows_v.at[b], out_hbm.at[pl.ds(base * Hp, GB * Hp)],
                so[b]).wait()

        def pair(t0, first):
            if not first:
                drain_o(0)
            h0 = fire_g(t0, 0)
            if not first:
                drain_o(1)
            h1 = fire_g(t0 + 1, 1)
            h0.wait()
            fire_o(t0, 0)
            h1.wait()
            fire_o(t0 + 1, 1)

        pair(0, True)

        def body(k, carry):
            pair(k * 2, False)
            return carry

        lax.fori_loop(1, K, body, 0)
        drain_o(0)
        drain_o(1)

    return gather_kernel


def kernel(chars, table):
    B, H = chars.shape
    Hp = (H + 7) // 8 * 8
    pad = (jnp.arange(B * (Hp - H), dtype=jnp.int32) * 131) % table.shape[0]
    idx = jnp.concatenate(
        [chars.astype(jnp.int32), pad.reshape(B, Hp - H)], axis=1).reshape(-1)
    out = _build(B, H)(idx, table)
    return out.reshape(B, Hp, D)[:, :H, :]


# speedup vs baseline: 8.1027x; 1.1490x over previous
import functools

import jax
import jax.numpy as jnp
from jax import lax
from jax.experimental import pallas as pl
from jax.experimental.pallas import tpu as pltpu
from jax.experimental.pallas import tpu_sc as plsc

D = 128
NW = 32
GB = 4


def _build(B, H):
    Hp = (H + 7) // 8 * 8
    rows_per_w = B // NW
    T = rows_per_w // GB
    K = T // 2

    mesh = plsc.VectorSubcoreMesh(core_axis_name="c", subcore_axis_name="s")

    @functools.partial(
        pl.kernel,
        mesh=mesh,
        out_type=jax.ShapeDtypeStruct((B, H, D), jnp.float32),
        scratch_types=[
            pltpu.VMEM((rows_per_w * Hp,), jnp.int32),
            pltpu.VMEM((2, GB * Hp, D), jnp.float32),
            pltpu.SemaphoreType.DMA,
            pltpu.SemaphoreType.DMA,
            pltpu.SemaphoreType.DMA,
            pltpu.SemaphoreType.DMA,
        ],
    )
    def gather_kernel(idx_hbm, table_hbm, out_hbm, idx_v, rows_v,
                      sg0, sg1, so0, so1):
        wid = lax.axis_index("s") * 2 + lax.axis_index("c")
        base = wid * rows_per_w
        sg = (sg0, sg1)
        so = (so0, so1)

        pltpu.sync_copy(idx_hbm.at[pl.ds(base * Hp, rows_per_w * Hp)], idx_v)

        def fire_g(t, b):
            return pltpu.async_copy(
                table_hbm.at[idx_v.at[pl.ds(t * GB * Hp, GB * Hp)]],
                rows_v.at[b], sg[b])

        def fire_o(t, b):
            for j in range(GB):
                pltpu.async_copy(
                    rows_v.at[b, pl.ds(j * Hp, H)],
                    out_hbm.at[base + t * GB + j], so[b])

        def drain_o(b):
            for j in range(GB):
                pltpu.make_async_copy(
                    rows_v.at[b, pl.ds(j * Hp, H)], out_hbm.at[base + j],
                    so[b]).wait()

        def pair(t0, first):
            if not first:
                drain_o(0)
            h0 = fire_g(t0, 0)
            if not first:
                drain_o(1)
            h1 = fire_g(t0 + 1, 1)
            h0.wait()
            fire_o(t0, 0)
            h1.wait()
            fire_o(t0 + 1, 1)

        pair(0, True)

        def body(k, carry):
            pair(k * 2, False)
            return carry

        lax.fori_loop(1, K, body, 0)
        drain_o(0)
        drain_o(1)

    return gather_kernel


def kernel(chars, table):
    B, H = chars.shape
    Hp = (H + 7) // 8 * 8
    pad = (jnp.arange(B * (Hp - H), dtype=jnp.int32) * 131) % table.shape[0]
    idx = jnp.concatenate(
        [chars.astype(jnp.int32), pad.reshape(B, Hp - H)], axis=1).reshape(-1)
    return _build(B, H)(idx, table)


# speedup vs baseline: 8.1177x; 1.0019x over previous
import functools

import jax
import jax.numpy as jnp
from jax import lax
from jax.experimental import pallas as pl
from jax.experimental.pallas import tpu as pltpu
from jax.experimental.pallas import tpu_sc as plsc

D = 128
NW = 32
GB = 4


def _build(B, H):
    Hp = (H + 7) // 8 * 8
    rows_per_w = B // NW
    T = rows_per_w // GB
    K = T // 2

    mesh = plsc.VectorSubcoreMesh(core_axis_name="c", subcore_axis_name="s")

    @functools.partial(
        pl.kernel,
        mesh=mesh,
        out_type=jax.ShapeDtypeStruct((B, H, D), jnp.float32),
        scratch_types=[
            pltpu.VMEM((rows_per_w * Hp,), jnp.int32),
            pltpu.VMEM((2, GB * Hp, D), jnp.float32),
            pltpu.SemaphoreType.DMA,
            pltpu.SemaphoreType.DMA,
            pltpu.SemaphoreType.DMA,
            pltpu.SemaphoreType.DMA,
        ],
    )
    def gather_kernel(idx_hbm, table_hbm, out_hbm, idx_v, rows_v,
                      sg0, sg1, so0, so1):
        wid = lax.axis_index("c") * (NW // 2) + lax.axis_index("s")
        base = wid * rows_per_w
        sg = (sg0, sg1)
        so = (so0, so1)

        pltpu.sync_copy(idx_hbm.at[pl.ds(base * Hp, rows_per_w * Hp)], idx_v)

        def fire_g(t, b):
            return pltpu.async_copy(
                table_hbm.at[idx_v.at[pl.ds(t * GB * Hp, GB * Hp)]],
                rows_v.at[b], sg[b])

        def fire_o(t, b):
            for j in range(GB):
                pltpu.async_copy(
                    rows_v.at[b, pl.ds(j * Hp, H)],
                    out_hbm.at[base + t * GB + j], so[b])

        def drain_o(b):
            for j in range(GB):
                pltpu.make_async_copy(
                    rows_v.at[b, pl.ds(j * Hp, H)], out_hbm.at[base + j],
                    so[b]).wait()

        def pair(t0, first):
            if not first:
                drain_o(0)
            h0 = fire_g(t0, 0)
            if not first:
                drain_o(1)
            h1 = fire_g(t0 + 1, 1)
            h0.wait()
            fire_o(t0, 0)
            h1.wait()
            fire_o(t0 + 1, 1)

        pair(0, True)

        def body(k, carry):
            pair(k * 2, False)
            return carry

        lax.fori_loop(1, K, body, 0)
        drain_o(0)
        drain_o(1)

    return gather_kernel


def kernel(chars, table):
    B, H = chars.shape
    Hp = (H + 7) // 8 * 8
    pad = (jnp.arange(B * (Hp - H), dtype=jnp.int32) * 131) % table.shape[0]
    idx = jnp.concatenate(
        [chars.astype(jnp.int32), pad.reshape(B, Hp - H)], axis=1).reshape(-1)
    return _build(B, H)(idx, table)
